# baseline (speedup 1.0000x reference)
import functools

import jax
import jax.numpy as jnp
from jax import lax
from jax.experimental import pallas as pl
from jax.experimental.pallas import tpu as pltpu
from jax.experimental.pallas import tpu_sc as plsc

N = 10000
D = 128
E = 320000
NC = 2
NS = 16
NW = NC * NS
CH = 128
CPT = 80
EPW = CPT * CH
EPAD = EPW * NW
NPAD = 10240
RPT = NPAD // NS

_mesh = plsc.VectorSubcoreMesh(core_axis_name="c", subcore_axis_name="s")


@functools.partial(
    pl.kernel,
    out_type=jax.ShapeDtypeStruct((NC * NPAD, D), jnp.float32),
    mesh=_mesh,
    scratch_types=[
        pltpu.VMEM((CH,), jnp.int32),
        pltpu.VMEM((CH,), jnp.int32),
        pltpu.VMEM((CH, D), jnp.float32),
        pltpu.VMEM_SHARED((NPAD, D), jnp.float32),
        pltpu.SemaphoreType.DMA,
    ],
)
def _sc_scatter(t_hbm, row_hbm, col_hbm, zero_hbm, out_hbm,
                ridx, cidx, rows, agg_sh, gsem):
    c = lax.axis_index("c")
    s = lax.axis_index("s")
    w = c * NS + s
    base = w * EPW

    pltpu.sync_copy(zero_hbm, agg_sh.at[pl.ds(s * RPT, RPT)])
    plsc.subcore_barrier()

    @pl.loop(0, CPT)
    def _chunk(g):
        off = base + g * CH
        pltpu.sync_copy(row_hbm.at[pl.ds(off, CH)], ridx)
        pltpu.sync_copy(col_hbm.at[pl.ds(off, CH)], cidx)
        pltpu.async_copy(t_hbm.at[ridx], rows, gsem).wait()
        pltpu.sync_copy(rows, agg_sh.at[cidx], add=True)

    plsc.subcore_barrier()
    r0 = s * RPT
    pltpu.sync_copy(agg_sh.at[pl.ds(r0, RPT)],
                    out_hbm.at[pl.ds(c * NPAD + r0, RPT)])


def _mm_body(x_ref, w_ref, b_ref, o_ref):
    o_ref[...] = lax.dot_general(
        x_ref[...], w_ref[...], (((1,), (1,)), ((), ())),
        preferred_element_type=jnp.float32) + b_ref[...]


def _comb_mm_body(t_ref, a0_ref, a1_ref, w_ref, b_ref, o_ref):
    x = jnp.maximum((t_ref[...] + a0_ref[...] + a1_ref[...]) * 0.5, 0.0)
    o_ref[...] = lax.dot_general(
        x, w_ref[...], (((1,), (1,)), ((), ())),
        preferred_element_type=jnp.float32) + b_ref[...]


def _final_body(t_ref, a0_ref, a1_ref, o_ref):
    o_ref[...] = (t_ref[...] + a0_ref[...] + a1_ref[...]) * 0.5


_BR = 1000


def _row_spec(br):
    return pl.BlockSpec((br, D), lambda i: (i, 0))


def _full_spec(shape):
    return pl.BlockSpec(shape, lambda i: (0,) * len(shape))


def _mm(x, w, b):
    return pl.pallas_call(
        _mm_body,
        grid=(N // _BR,),
        in_specs=[_row_spec(_BR), _full_spec((D, D)), _full_spec((1, D))],
        out_specs=_row_spec(_BR),
        out_shape=jax.ShapeDtypeStruct((N, D), jnp.float32),
    )(x, w, b)


def _comb_mm(t, a0, a1, w, b):
    return pl.pallas_call(
        _comb_mm_body,
        grid=(N // _BR,),
        in_specs=[_row_spec(_BR)] * 3 + [_full_spec((D, D)), _full_spec((1, D))],
        out_specs=_row_spec(_BR),
        out_shape=jax.ShapeDtypeStruct((N, D), jnp.float32),
    )(t, a0, a1, w, b)


def _final(t, a0, a1):
    return pl.pallas_call(
        _final_body,
        grid=(N // _BR,),
        in_specs=[_row_spec(_BR)] * 3,
        out_specs=_row_spec(_BR),
        out_shape=jax.ShapeDtypeStruct((N, D), jnp.float32),
    )(t, a0, a1)


def kernel(node_features, edge_index, w0, b0, w1, b1, hidden_dim):
    del hidden_dim
    row = edge_index[0]
    col = edge_index[1]
    pad = EPAD - E
    row_p = jnp.concatenate([row, jnp.zeros((pad,), jnp.int32)])
    col_p = jnp.concatenate([col, jnp.full((pad,), N, jnp.int32)])
    zero_tile = jnp.zeros((RPT, D), jnp.float32)

    t0 = _mm(node_features, w0[0], b0)
    agg0 = _sc_scatter(t0, row_p, col_p, zero_tile)
    t1 = _comb_mm(t0, agg0[:N], agg0[NPAD:NPAD + N], w1[0], b1)
    agg1 = _sc_scatter(t1, row_p, col_p, zero_tile)
    return _final(t1, agg1[:N], agg1[NPAD:NPAD + N])


# speedup vs baseline: 2.2870x; 2.2870x over previous
import functools

import jax
import jax.numpy as jnp
from jax import lax
from jax.experimental import pallas as pl
from jax.experimental.pallas import tpu as pltpu
from jax.experimental.pallas import tpu_sc as plsc

N = 10000
D = 128
E = 320000
NC = 2
NS = 16
NW = NC * NS
CH = 128
CPT = 80
EPW = CPT * CH
EPAD = EPW * NW
NPAD = 10240
RPT = NPAD // NS

_mesh = plsc.VectorSubcoreMesh(core_axis_name="c", subcore_axis_name="s")


@functools.partial(
    pl.kernel,
    out_type=jax.ShapeDtypeStruct((NC * NPAD, D), jnp.float32),
    mesh=_mesh,
    scratch_types=[
        pltpu.VMEM((CH,), jnp.int32),
        pltpu.VMEM((CH,), jnp.int32),
        pltpu.VMEM((CH, D), jnp.float32),
        pltpu.VMEM_SHARED((NPAD, D), jnp.float32),
        pltpu.SemaphoreType.DMA,
    ],
)
def _sc_scatter(t_hbm, row_hbm, col_hbm, zero_hbm, out_hbm,
                ridx, cidx, rows, agg_sh, gsem):
    c = lax.axis_index("c")
    s = lax.axis_index("s")
    w = c * NS + s
    base = w * EPW

    pltpu.sync_copy(zero_hbm, agg_sh.at[pl.ds(s * RPT, RPT)])
    plsc.subcore_barrier()

    @pl.loop(0, CPT)
    def _chunk(g):
        off = base + g * CH
        pltpu.sync_copy(row_hbm.at[pl.ds(off, CH)], ridx)
        pltpu.sync_copy(col_hbm.at[pl.ds(off, CH)], cidx)
        pltpu.async_copy(t_hbm.at[ridx], rows, gsem).wait()
        pltpu.sync_copy(rows, agg_sh.at[cidx], add=True)

    plsc.subcore_barrier()
    r0 = s * RPT
    pltpu.sync_copy(agg_sh.at[pl.ds(r0, RPT)],
                    out_hbm.at[pl.ds(c * NPAD + r0, RPT)])


def _mm_body(x_ref, w_ref, b_ref, o_ref):
    o_ref[...] = lax.dot_general(
        x_ref[...], w_ref[...], (((1,), (1,)), ((), ())),
        preferred_element_type=jnp.float32) + b_ref[...]


def _comb_mm_body(t_ref, a0_ref, a1_ref, w_ref, b_ref, o_ref):
    x = jnp.maximum((t_ref[...] + a0_ref[...] + a1_ref[...]) * 0.5, 0.0)
    o_ref[...] = lax.dot_general(
        x, w_ref[...], (((1,), (1,)), ((), ())),
        preferred_element_type=jnp.float32) + b_ref[...]


def _final_body(t_ref, a0_ref, a1_ref, o_ref):
    o_ref[...] = (t_ref[...] + a0_ref[...] + a1_ref[...]) * 0.5


_BR = 1000


def _row_spec(br):
    return pl.BlockSpec((br, D), lambda i: (i, 0))


def _full_spec(shape):
    return pl.BlockSpec(shape, lambda i: (0,) * len(shape))


def _mm(x, w, b):
    return pl.pallas_call(
        _mm_body,
        grid=(N // _BR,),
        in_specs=[_row_spec(_BR), _full_spec((D, D)), _full_spec((1, D))],
        out_specs=_row_spec(_BR),
        out_shape=jax.ShapeDtypeStruct((N, D), jnp.float32),
    )(x, w, b)


def _comb_mm(t, a0, a1, w, b):
    return pl.pallas_call(
        _comb_mm_body,
        grid=(N // _BR,),
        in_specs=[_row_spec(_BR)] * 3 + [_full_spec((D, D)), _full_spec((1, D))],
        out_specs=_row_spec(_BR),
        out_shape=jax.ShapeDtypeStruct((N, D), jnp.float32),
    )(t, a0, a1, w, b)


def _final(t, a0, a1):
    return pl.pallas_call(
        _final_body,
        grid=(N // _BR,),
        in_specs=[_row_spec(_BR)] * 3,
        out_specs=_row_spec(_BR),
        out_shape=jax.ShapeDtypeStruct((N, D), jnp.float32),
    )(t, a0, a1)


def kernel(node_features, edge_index, w0, b0, w1, b1, hidden_dim):
    del hidden_dim
    row = edge_index[0]
    col = edge_index[1]
    pad = EPAD - E
    ar = jnp.arange(pad, dtype=jnp.int32)
    row_p = jnp.concatenate([row, ar % N])
    col_p = jnp.concatenate([col, N + ar % (NPAD - N)])
    zero_tile = jnp.zeros((RPT, D), jnp.float32)

    t0 = _mm(node_features, w0[0], b0)
    agg0 = _sc_scatter(t0, row_p, col_p, zero_tile)
    t1 = _comb_mm(t0, agg0[:N], agg0[NPAD:NPAD + N], w1[0], b1)
    agg1 = _sc_scatter(t1, row_p, col_p, zero_tile)
    return _final(t1, agg1[:N], agg1[NPAD:NPAD + N])


# speedup vs baseline: 3.0644x; 1.3399x over previous
import functools

import jax
import jax.numpy as jnp
from jax import lax
from jax.experimental import pallas as pl
from jax.experimental.pallas import tpu as pltpu
from jax.experimental.pallas import tpu_sc as plsc

N = 10000
D = 128
E = 320000
NC = 2
NS = 16
NW = NC * NS
CH = 128
NB = 2
CPT = 80
EPW = CPT * CH
EPAD = EPW * NW
NPAD = 10240
RPT = NPAD // NS

_mesh = plsc.VectorSubcoreMesh(core_axis_name="c", subcore_axis_name="s")


@functools.partial(
    pl.kernel,
    out_type=jax.ShapeDtypeStruct((NC * NPAD, D), jnp.float32),
    mesh=_mesh,
    scratch_types=[
        [pltpu.VMEM((CH,), jnp.int32)] * NB,
        [pltpu.VMEM((CH,), jnp.int32)] * NB,
        [pltpu.VMEM((CH, D), jnp.float32)] * NB,
        pltpu.VMEM_SHARED((NPAD, D), jnp.float32),
        [pltpu.SemaphoreType.DMA] * NB,
    ],
)
def _sc_scatter(t_hbm, row_hbm, col_hbm, zero_hbm, out_hbm,
                ridx, cidx, rows, agg_sh, gsem):
    c = lax.axis_index("c")
    s = lax.axis_index("s")
    w = c * NS + s
    base = w * EPW

    def load_idx(g, b):
        off = base + g * CH
        pltpu.sync_copy(row_hbm.at[pl.ds(off, CH)], ridx[b])
        pltpu.sync_copy(col_hbm.at[pl.ds(off, CH)], cidx[b])

    pltpu.sync_copy(zero_hbm, agg_sh.at[pl.ds(s * RPT, RPT)])
    plsc.subcore_barrier()

    @pl.loop(0, CPT // 2)
    def _pair(h):
        g = 2 * h
        load_idx(g, 0)
        da = pltpu.async_copy(t_hbm.at[ridx[0]], rows[0], gsem[0])
        load_idx(g + 1, 1)
        db = pltpu.async_copy(t_hbm.at[ridx[1]], rows[1], gsem[1])
        da.wait()
        pltpu.sync_copy(rows[0], agg_sh.at[cidx[0]], add=True)
        db.wait()
        pltpu.sync_copy(rows[1], agg_sh.at[cidx[1]], add=True)

    plsc.subcore_barrier()
    r0 = s * RPT
    pltpu.sync_copy(agg_sh.at[pl.ds(r0, RPT)],
                    out_hbm.at[pl.ds(c * NPAD + r0, RPT)])


def _mm_body(x_ref, w_ref, b_ref, o_ref):
    o_ref[...] = lax.dot_general(
        x_ref[...], w_ref[...], (((1,), (1,)), ((), ())),
        preferred_element_type=jnp.float32) + b_ref[...]


def _comb_mm_body(t_ref, a0_ref, a1_ref, w_ref, b_ref, o_ref):
    x = jnp.maximum((t_ref[...] + a0_ref[...] + a1_ref[...]) * 0.5, 0.0)
    o_ref[...] = lax.dot_general(
        x, w_ref[...], (((1,), (1,)), ((), ())),
        preferred_element_type=jnp.float32) + b_ref[...]


def _final_body(t_ref, a0_ref, a1_ref, o_ref):
    o_ref[...] = (t_ref[...] + a0_ref[...] + a1_ref[...]) * 0.5


_BR = 1000


def _row_spec(br):
    return pl.BlockSpec((br, D), lambda i: (i, 0))


def _full_spec(shape):
    return pl.BlockSpec(shape, lambda i: (0,) * len(shape))


def _mm(x, w, b):
    return pl.pallas_call(
        _mm_body,
        grid=(N // _BR,),
        in_specs=[_row_spec(_BR), _full_spec((D, D)), _full_spec((1, D))],
        out_specs=_row_spec(_BR),
        out_shape=jax.ShapeDtypeStruct((N, D), jnp.float32),
    )(x, w, b)


def _comb_mm(t, a0, a1, w, b):
    return pl.pallas_call(
        _comb_mm_body,
        grid=(N // _BR,),
        in_specs=[_row_spec(_BR)] * 3 + [_full_spec((D, D)), _full_spec((1, D))],
        out_specs=_row_spec(_BR),
        out_shape=jax.ShapeDtypeStruct((N, D), jnp.float32),
    )(t, a0, a1, w, b)


def _final(t, a0, a1):
    return pl.pallas_call(
        _final_body,
        grid=(N // _BR,),
        in_specs=[_row_spec(_BR)] * 3,
        out_specs=_row_spec(_BR),
        out_shape=jax.ShapeDtypeStruct((N, D), jnp.float32),
    )(t, a0, a1)


def kernel(node_features, edge_index, w0, b0, w1, b1, hidden_dim):
    del hidden_dim
    row = edge_index[0]
    col = edge_index[1]
    pad = EPAD - E
    ar = jnp.arange(pad, dtype=jnp.int32)
    row_p = jnp.concatenate([row, ar % N])
    col_p = jnp.concatenate([col, N + ar % (NPAD - N)])
    zero_tile = jnp.zeros((RPT, D), jnp.float32)

    t0 = _mm(node_features, w0[0], b0)
    agg0 = _sc_scatter(t0, row_p, col_p, zero_tile)
    t1 = _comb_mm(t0, agg0[:N], agg0[NPAD:NPAD + N], w1[0], b1)
    agg1 = _sc_scatter(t1, row_p, col_p, zero_tile)
    return _final(t1, agg1[:N], agg1[NPAD:NPAD + N])


# speedup vs baseline: 3.0653x; 1.0003x over previous
import functools

import jax
import jax.numpy as jnp
from jax import lax
from jax.experimental import pallas as pl
from jax.experimental.pallas import tpu as pltpu
from jax.experimental.pallas import tpu_sc as plsc

N = 10000
D = 128
E = 320000
NC = 2
NS = 16
NW = NC * NS
CH = 128
NB = 2
CPT = 80
EPW = CPT * CH
EPAD = EPW * NW
NPAD = 10240
RPT = NPAD // NS

_mesh = plsc.VectorSubcoreMesh(core_axis_name="c", subcore_axis_name="s")


@functools.partial(
    pl.kernel,
    out_type=jax.ShapeDtypeStruct((NC * NPAD, D), jnp.float32),
    mesh=_mesh,
    scratch_types=[
        [pltpu.VMEM((CH,), jnp.int32)] * NB,
        [pltpu.VMEM((CH,), jnp.int32)] * NB,
        [pltpu.VMEM((CH, D), jnp.float32)] * NB,
        pltpu.VMEM_SHARED((NPAD, D), jnp.float32),
        [pltpu.SemaphoreType.DMA] * NB,
        [pltpu.SemaphoreType.DMA] * NB,
    ],
)
def _sc_scatter(t_hbm, row_hbm, col_hbm, zero_hbm, out_hbm,
                ridx, cidx, rows, agg_sh, gsem, ssem):
    c = lax.axis_index("c")
    s = lax.axis_index("s")
    w = c * NS + s
    base = w * EPW

    def load_idx(g, b):
        off = base + g * CH
        pltpu.sync_copy(row_hbm.at[pl.ds(off, CH)], ridx[b])
        pltpu.sync_copy(col_hbm.at[pl.ds(off, CH)], cidx[b])

    pltpu.sync_copy(zero_hbm, agg_sh.at[pl.ds(s * RPT, RPT)])
    plsc.subcore_barrier()

    @pl.loop(0, CPT // 2)
    def _pair(h):
        g = 2 * h
        load_idx(g, 0)
        da = pltpu.async_copy(t_hbm.at[ridx[0]], rows[0], gsem[0])
        load_idx(g + 1, 1)
        db = pltpu.async_copy(t_hbm.at[ridx[1]], rows[1], gsem[1])
        da.wait()
        sa = pltpu.async_copy(rows[0], agg_sh.at[cidx[0]], ssem[0], add=True)
        db.wait()
        sb = pltpu.async_copy(rows[1], agg_sh.at[cidx[1]], ssem[1], add=True)
        sa.wait()
        sb.wait()

    plsc.subcore_barrier()
    r0 = s * RPT
    pltpu.sync_copy(agg_sh.at[pl.ds(r0, RPT)],
                    out_hbm.at[pl.ds(c * NPAD + r0, RPT)])


def _mm_body(x_ref, w_ref, b_ref, o_ref):
    o_ref[...] = lax.dot_general(
        x_ref[...], w_ref[...], (((1,), (1,)), ((), ())),
        preferred_element_type=jnp.float32) + b_ref[...]


def _comb_mm_body(t_ref, a0_ref, a1_ref, w_ref, b_ref, o_ref):
    x = jnp.maximum((t_ref[...] + a0_ref[...] + a1_ref[...]) * 0.5, 0.0)
    o_ref[...] = lax.dot_general(
        x, w_ref[...], (((1,), (1,)), ((), ())),
        preferred_element_type=jnp.float32) + b_ref[...]


def _final_body(t_ref, a0_ref, a1_ref, o_ref):
    o_ref[...] = (t_ref[...] + a0_ref[...] + a1_ref[...]) * 0.5


_BR = 1000


def _row_spec(br):
    return pl.BlockSpec((br, D), lambda i: (i, 0))


def _full_spec(shape):
    return pl.BlockSpec(shape, lambda i: (0,) * len(shape))


def _mm(x, w, b):
    return pl.pallas_call(
        _mm_body,
        grid=(N // _BR,),
        in_specs=[_row_spec(_BR), _full_spec((D, D)), _full_spec((1, D))],
        out_specs=_row_spec(_BR),
        out_shape=jax.ShapeDtypeStruct((N, D), jnp.float32),
    )(x, w, b)


def _comb_mm(t, a0, a1, w, b):
    return pl.pallas_call(
        _comb_mm_body,
        grid=(N // _BR,),
        in_specs=[_row_spec(_BR)] * 3 + [_full_spec((D, D)), _full_spec((1, D))],
        out_specs=_row_spec(_BR),
        out_shape=jax.ShapeDtypeStruct((N, D), jnp.float32),
    )(t, a0, a1, w, b)


def _final(t, a0, a1):
    return pl.pallas_call(
        _final_body,
        grid=(N // _BR,),
        in_specs=[_row_spec(_BR)] * 3,
        out_specs=_row_spec(_BR),
        out_shape=jax.ShapeDtypeStruct((N, D), jnp.float32),
    )(t, a0, a1)


def kernel(node_features, edge_index, w0, b0, w1, b1, hidden_dim):
    del hidden_dim
    row = edge_index[0]
    col = edge_index[1]
    pad = EPAD - E
    ar = jnp.arange(pad, dtype=jnp.int32)
    row_p = jnp.concatenate([row, ar % N])
    col_p = jnp.concatenate([col, N + ar % (NPAD - N)])
    zero_tile = jnp.zeros((RPT, D), jnp.float32)

    t0 = _mm(node_features, w0[0], b0)
    agg0 = _sc_scatter(t0, row_p, col_p, zero_tile)
    t1 = _comb_mm(t0, agg0[:N], agg0[NPAD:NPAD + N], w1[0], b1)
    agg1 = _sc_scatter(t1, row_p, col_p, zero_tile)
    return _final(t1, agg1[:N], agg1[NPAD:NPAD + N])


# speedup vs baseline: 3.3716x; 1.0999x over previous
import functools

import jax
import jax.numpy as jnp
from jax import lax
from jax.experimental import pallas as pl
from jax.experimental.pallas import tpu as pltpu
from jax.experimental.pallas import tpu_sc as plsc

N = 10000
D = 128
E = 320000
NC = 2
NS = 16
NW = NC * NS
CH = 128
NB = 3
CPT = 81
EPW = CPT * CH
EPAD = EPW * NW
NPAD = 10112
RPT = NPAD // NS

_mesh = plsc.VectorSubcoreMesh(core_axis_name="c", subcore_axis_name="s")


@functools.partial(
    pl.kernel,
    out_type=jax.ShapeDtypeStruct((NC * NPAD, D), jnp.float32),
    mesh=_mesh,
    scratch_types=[
        [pltpu.VMEM((CH,), jnp.int32)] * NB,
        [pltpu.VMEM((CH,), jnp.int32)] * NB,
        [pltpu.VMEM((CH, D), jnp.float32)] * NB,
        pltpu.VMEM_SHARED((NPAD, D), jnp.float32),
        [pltpu.SemaphoreType.DMA] * NB,
        [pltpu.SemaphoreType.DMA] * NB,
        [pltpu.SemaphoreType.DMA] * NB,
    ],
)
def _sc_scatter(t_hbm, row_hbm, col_hbm, zero_hbm, out_hbm,
                ridx, cidx, rows, agg_sh, gsem, ssem, isem):
    c = lax.axis_index("c")
    s = lax.axis_index("s")
    w = c * NS + s
    base = w * EPW

    pltpu.sync_copy(zero_hbm, agg_sh.at[pl.ds(s * RPT, RPT)])
    plsc.subcore_barrier()

    @pl.loop(0, CPT // NB)
    def _group(h):
        g0 = h * NB
        iloads = []
        for b in range(NB):
            off = base + (g0 + b) * CH
            iloads.append((
                pltpu.async_copy(row_hbm.at[pl.ds(off, CH)], ridx[b], isem[b]),
                pltpu.async_copy(col_hbm.at[pl.ds(off, CH)], cidx[b], isem[b]),
            ))
        gathers = []
        for b in range(NB):
            iloads[b][0].wait()
            iloads[b][1].wait()
            gathers.append(
                pltpu.async_copy(t_hbm.at[ridx[b]], rows[b], gsem[b]))
        scatters = []
        for b in range(NB):
            gathers[b].wait()
            scatters.append(pltpu.async_copy(
                rows[b], agg_sh.at[cidx[b]], ssem[b], add=True))
        for sd in scatters:
            sd.wait()

    plsc.subcore_barrier()
    r0 = s * RPT
    pltpu.sync_copy(agg_sh.at[pl.ds(r0, RPT)],
                    out_hbm.at[pl.ds(c * NPAD + r0, RPT)])


def _mm_body(x_ref, w_ref, b_ref, o_ref):
    o_ref[...] = lax.dot_general(
        x_ref[...], w_ref[...], (((1,), (1,)), ((), ())),
        preferred_element_type=jnp.float32) + b_ref[...]


def _comb_mm_body(t_ref, a0_ref, a1_ref, w_ref, b_ref, o_ref):
    x = jnp.maximum((t_ref[...] + a0_ref[...] + a1_ref[...]) * 0.5, 0.0)
    o_ref[...] = lax.dot_general(
        x, w_ref[...], (((1,), (1,)), ((), ())),
        preferred_element_type=jnp.float32) + b_ref[...]


def _final_body(t_ref, a0_ref, a1_ref, o_ref):
    o_ref[...] = (t_ref[...] + a0_ref[...] + a1_ref[...]) * 0.5


_BR = 1000


def _row_spec(br):
    return pl.BlockSpec((br, D), lambda i: (i, 0))


def _full_spec(shape):
    return pl.BlockSpec(shape, lambda i: (0,) * len(shape))


def _mm(x, w, b):
    return pl.pallas_call(
        _mm_body,
        grid=(N // _BR,),
        in_specs=[_row_spec(_BR), _full_spec((D, D)), _full_spec((1, D))],
        out_specs=_row_spec(_BR),
        out_shape=jax.ShapeDtypeStruct((N, D), jnp.float32),
    )(x, w, b)


def _comb_mm(t, a0, a1, w, b):
    return pl.pallas_call(
        _comb_mm_body,
        grid=(N // _BR,),
        in_specs=[_row_spec(_BR)] * 3 + [_full_spec((D, D)), _full_spec((1, D))],
        out_specs=_row_spec(_BR),
        out_shape=jax.ShapeDtypeStruct((N, D), jnp.float32),
    )(t, a0, a1, w, b)


def _final(t, a0, a1):
    return pl.pallas_call(
        _final_body,
        grid=(N // _BR,),
        in_specs=[_row_spec(_BR)] * 3,
        out_specs=_row_spec(_BR),
        out_shape=jax.ShapeDtypeStruct((N, D), jnp.float32),
    )(t, a0, a1)


def kernel(node_features, edge_index, w0, b0, w1, b1, hidden_dim):
    del hidden_dim
    row = edge_index[0]
    col = edge_index[1]
    pad = EPAD - E
    ar = jnp.arange(pad, dtype=jnp.int32)
    row_p = jnp.concatenate([row, ar % N])
    col_p = jnp.concatenate([col, N + ar % (NPAD - N)])
    zero_tile = jnp.zeros((RPT, D), jnp.float32)

    t0 = _mm(node_features, w0[0], b0)
    agg0 = _sc_scatter(t0, row_p, col_p, zero_tile)
    t1 = _comb_mm(t0, agg0[:N], agg0[NPAD:NPAD + N], w1[0], b1)
    agg1 = _sc_scatter(t1, row_p, col_p, zero_tile)
    return _final(t1, agg1[:N], agg1[NPAD:NPAD + N])


# speedup vs baseline: 3.6230x; 1.0746x over previous
import functools

import jax
import jax.numpy as jnp
from jax import lax
from jax.experimental import pallas as pl
from jax.experimental.pallas import tpu as pltpu
from jax.experimental.pallas import tpu_sc as plsc

N = 10000
D = 128
E = 320000
NC = 2
NS = 16
NW = NC * NS
CH = 128
NB = 3
CPT = 81
EPW = CPT * CH
EPAD = EPW * NW
NPAD = 10112
RPT = NPAD // NS

_mesh = plsc.VectorSubcoreMesh(core_axis_name="c", subcore_axis_name="s")


@functools.partial(
    pl.kernel,
    out_type=jax.ShapeDtypeStruct((NC * NPAD, D), jnp.float32),
    mesh=_mesh,
    scratch_types=[
        [pltpu.VMEM((CH,), jnp.int32)] * NB,
        [pltpu.VMEM((CH,), jnp.int32)] * NB,
        [pltpu.VMEM((CH, D), jnp.float32)] * NB,
        pltpu.VMEM_SHARED((NPAD, D), jnp.float32),
        [pltpu.SemaphoreType.DMA] * NB,
        [pltpu.SemaphoreType.DMA] * NB,
        [pltpu.SemaphoreType.DMA] * NB,
    ],
)
def _sc_scatter(t_hbm, row_hbm, col_hbm, zero_hbm, out_hbm,
                ridx, cidx, rows, agg_sh, gsem, ssem, isem):
    c = lax.axis_index("c")
    s = lax.axis_index("s")
    w = c * NS + s
    base = w * EPW

    pltpu.sync_copy(zero_hbm, agg_sh.at[pl.ds(s * RPT, RPT)])
    plsc.subcore_barrier()

    def drain_scatter(b):
        pltpu.make_async_copy(rows[b], agg_sh.at[cidx[b]], ssem[b]).wait()

    @pl.loop(0, CPT // NB)
    def _group(h):
        g0 = h * NB
        iloads = []
        for b in range(NB):
            @pl.when(h > 0)
            def _(b=b):
                drain_scatter(b)

            off = base + (g0 + b) * CH
            iloads.append((
                pltpu.async_copy(row_hbm.at[pl.ds(off, CH)], ridx[b], isem[b]),
                pltpu.async_copy(col_hbm.at[pl.ds(off, CH)], cidx[b], isem[b]),
            ))
        gathers = []
        for b in range(NB):
            iloads[b][0].wait()
            iloads[b][1].wait()
            gathers.append(
                pltpu.async_copy(t_hbm.at[ridx[b]], rows[b], gsem[b]))
        for b in range(NB):
            gathers[b].wait()
            pltpu.async_copy(rows[b], agg_sh.at[cidx[b]], ssem[b], add=True)

    for b in range(NB):
        drain_scatter(b)

    plsc.subcore_barrier()
    r0 = s * RPT
    pltpu.sync_copy(agg_sh.at[pl.ds(r0, RPT)],
                    out_hbm.at[pl.ds(c * NPAD + r0, RPT)])


def _mm_body(x_ref, w_ref, b_ref, o_ref):
    o_ref[...] = lax.dot_general(
        x_ref[...], w_ref[...], (((1,), (1,)), ((), ())),
        preferred_element_type=jnp.float32) + b_ref[...]


def _comb_mm_body(t_ref, a0_ref, a1_ref, w_ref, b_ref, o_ref):
    x = jnp.maximum((t_ref[...] + a0_ref[...] + a1_ref[...]) * 0.5, 0.0)
    o_ref[...] = lax.dot_general(
        x, w_ref[...], (((1,), (1,)), ((), ())),
        preferred_element_type=jnp.float32) + b_ref[...]


def _final_body(t_ref, a0_ref, a1_ref, o_ref):
    o_ref[...] = (t_ref[...] + a0_ref[...] + a1_ref[...]) * 0.5


_BR = 1000


def _row_spec(br):
    return pl.BlockSpec((br, D), lambda i: (i, 0))


def _full_spec(shape):
    return pl.BlockSpec(shape, lambda i: (0,) * len(shape))


def _mm(x, w, b):
    return pl.pallas_call(
        _mm_body,
        grid=(N // _BR,),
        in_specs=[_row_spec(_BR), _full_spec((D, D)), _full_spec((1, D))],
        out_specs=_row_spec(_BR),
        out_shape=jax.ShapeDtypeStruct((N, D), jnp.float32),
    )(x, w, b)


def _comb_mm(t, a0, a1, w, b):
    return pl.pallas_call(
        _comb_mm_body,
        grid=(N // _BR,),
        in_specs=[_row_spec(_BR)] * 3 + [_full_spec((D, D)), _full_spec((1, D))],
        out_specs=_row_spec(_BR),
        out_shape=jax.ShapeDtypeStruct((N, D), jnp.float32),
    )(t, a0, a1, w, b)


def _final(t, a0, a1):
    return pl.pallas_call(
        _final_body,
        grid=(N // _BR,),
        in_specs=[_row_spec(_BR)] * 3,
        out_specs=_row_spec(_BR),
        out_shape=jax.ShapeDtypeStruct((N, D), jnp.float32),
    )(t, a0, a1)


def kernel(node_features, edge_index, w0, b0, w1, b1, hidden_dim):
    del hidden_dim
    row = edge_index[0]
    col = edge_index[1]
    pad = EPAD - E
    ar = jnp.arange(pad, dtype=jnp.int32)
    row_p = jnp.concatenate([row, ar % N])
    col_p = jnp.concatenate([col, N + ar % (NPAD - N)])
    zero_tile = jnp.zeros((RPT, D), jnp.float32)

    t0 = _mm(node_features, w0[0], b0)
    agg0 = _sc_scatter(t0, row_p, col_p, zero_tile)
    t1 = _comb_mm(t0, agg0[:N], agg0[NPAD:NPAD + N], w1[0], b1)
    agg1 = _sc_scatter(t1, row_p, col_p, zero_tile)
    return _final(t1, agg1[:N], agg1[NPAD:NPAD + N])


# speedup vs baseline: 3.7310x; 1.0298x over previous
import functools

import jax
import jax.numpy as jnp
from jax import lax
from jax.experimental import pallas as pl
from jax.experimental.pallas import tpu as pltpu
from jax.experimental.pallas import tpu_sc as plsc

N = 10000
D = 128
E = 320000
NC = 2
NS = 16
NW = NC * NS
CH = 128
NB = 3
CPT = 81
EPW = CPT * CH
EPAD = EPW * NW
NPAD = 10112
RPT = NPAD // NS

_mesh = plsc.VectorSubcoreMesh(core_axis_name="c", subcore_axis_name="s")


@functools.partial(
    pl.kernel,
    out_type=jax.ShapeDtypeStruct((NC * NPAD, D), jnp.float32),
    mesh=_mesh,
    scratch_types=[
        [pltpu.VMEM((CH,), jnp.int32)] * NB,
        [pltpu.VMEM((CH,), jnp.int32)] * NB,
        [pltpu.VMEM((CH, D), jnp.float32)] * NB,
        pltpu.VMEM_SHARED((NPAD, D), jnp.float32),
        [pltpu.SemaphoreType.DMA] * NB,
        [pltpu.SemaphoreType.DMA] * NB,
        [pltpu.SemaphoreType.DMA] * NB,
    ],
)
def _sc_scatter(t_hbm, row_hbm, col_hbm, zero_hbm, out_hbm,
                ridx, cidx, rows, agg_sh, gsem, ssem, isem):
    c = lax.axis_index("c")
    s = lax.axis_index("s")
    w = c * NS + s
    base = w * EPW

    pltpu.sync_copy(zero_hbm, agg_sh.at[pl.ds(s * RPT, RPT)])
    plsc.subcore_barrier()

    def drain_scatter(b):
        pltpu.make_async_copy(rows[b], agg_sh.at[cidx[b]], ssem[b]).wait()

    @pl.loop(0, CPT // NB)
    def _group(h):
        g0 = h * NB
        iloads = []
        for b in range(NB):
            @pl.when(h > 0)
            def _(b=b):
                drain_scatter(b)

            off = base + (g0 + b) * CH
            iloads.append((
                pltpu.async_copy(row_hbm.at[pl.ds(off, CH)], ridx[b], isem[b]),
                pltpu.async_copy(col_hbm.at[pl.ds(off, CH)], cidx[b], isem[b]),
            ))
        gathers = []
        for b in range(NB):
            iloads[b][0].wait()
            iloads[b][1].wait()
            gathers.append(
                pltpu.async_copy(t_hbm.at[ridx[b]], rows[b], gsem[b]))
        for b in range(NB):
            gathers[b].wait()
            pltpu.async_copy(rows[b], agg_sh.at[cidx[b]], ssem[b], add=True)

    for b in range(NB):
        drain_scatter(b)

    plsc.subcore_barrier()
    r0 = s * RPT
    pltpu.sync_copy(agg_sh.at[pl.ds(r0, RPT)],
                    out_hbm.at[pl.ds(c * NPAD + r0, RPT)])


def _mm_body(x_ref, w_ref, b_ref, o_ref):
    o_ref[...] = lax.dot_general(
        x_ref[...], w_ref[...], (((1,), (1,)), ((), ())),
        preferred_element_type=jnp.float32) + b_ref[...]


def _comb_mm_body(t_ref, a0_ref, a1_ref, w_ref, b_ref, o_ref):
    x = jnp.maximum((t_ref[...] + a0_ref[0] + a1_ref[0]) * 0.5, 0.0)
    o_ref[...] = lax.dot_general(
        x, w_ref[...], (((1,), (1,)), ((), ())),
        preferred_element_type=jnp.float32) + b_ref[...]


def _final_body(t_ref, a0_ref, a1_ref, o_ref):
    o_ref[...] = (t_ref[...] + a0_ref[0] + a1_ref[0]) * 0.5


_BR = 1000


def _row_spec(br):
    return pl.BlockSpec((br, D), lambda i: (i, 0))


def _agg_spec(br, core):
    return pl.BlockSpec((1, br, D), lambda i, core=core: (core, i, 0))


def _full_spec(shape):
    return pl.BlockSpec(shape, lambda i: (0,) * len(shape))


def _mm(x, w, b):
    return pl.pallas_call(
        _mm_body,
        grid=(N // _BR,),
        in_specs=[_row_spec(_BR), _full_spec((D, D)), _full_spec((1, D))],
        out_specs=_row_spec(_BR),
        out_shape=jax.ShapeDtypeStruct((N, D), jnp.float32),
    )(x, w, b)


def _comb_mm(t, agg, w, b):
    return pl.pallas_call(
        _comb_mm_body,
        grid=(N // _BR,),
        in_specs=[_row_spec(_BR), _agg_spec(_BR, 0), _agg_spec(_BR, 1),
                  _full_spec((D, D)), _full_spec((1, D))],
        out_specs=_row_spec(_BR),
        out_shape=jax.ShapeDtypeStruct((N, D), jnp.float32),
    )(t, agg, agg, w, b)


def _final(t, agg):
    return pl.pallas_call(
        _final_body,
        grid=(N // _BR,),
        in_specs=[_row_spec(_BR), _agg_spec(_BR, 0), _agg_spec(_BR, 1)],
        out_specs=_row_spec(_BR),
        out_shape=jax.ShapeDtypeStruct((N, D), jnp.float32),
    )(t, agg, agg)


def kernel(node_features, edge_index, w0, b0, w1, b1, hidden_dim):
    del hidden_dim
    row = edge_index[0]
    col = edge_index[1]
    pad = EPAD - E
    ar = jnp.arange(pad, dtype=jnp.int32)
    row_p = jnp.concatenate([row, ar % N])
    col_p = jnp.concatenate([col, N + ar % (NPAD - N)])
    zero_tile = jnp.zeros((RPT, D), jnp.float32)

    t0 = _mm(node_features, w0[0], b0)
    agg0 = _sc_scatter(t0, row_p, col_p, zero_tile).reshape(NC, NPAD, D)
    t1 = _comb_mm(t0, agg0, w1[0], b1)
    agg1 = _sc_scatter(t1, row_p, col_p, zero_tile).reshape(NC, NPAD, D)
    return _final(t1, agg1)
